# Initial kernel scaffold; baseline (speedup 1.0000x reference)
import jax
import jax.numpy as jnp
from jax.experimental import pallas as pl


def kernel(item_embeddings, batch_data):
    raise NotImplementedError("write your pallas kernel here")


# speedup vs baseline: 1.6851x; 1.6851x over previous
import functools

import jax
import jax.numpy as jnp
from jax import lax
from jax.experimental import pallas as pl
from jax.experimental.pallas import tpu as pltpu
from jax.experimental.pallas import tpu_sc as plsc

VOCAB = 1000000
EMBED_DIM = 64
BATCH = 16384
HIST_LEN = 50

NUM_IDX = BATCH * HIST_LEN
NUM_WORKERS = 32
PER_WORKER = NUM_IDX // NUM_WORKERS
CHUNK = 128
NCHUNK = PER_WORKER // CHUNK


def _gather_kernel(table, idx_hbm, out, idx_v, rows_v, gsem):
    wid = lax.axis_index("s") * 2 + lax.axis_index("c")
    base = wid * PER_WORKER
    pltpu.sync_copy(idx_hbm.at[wid], idx_v)

    @pl.loop(0, NCHUNK)
    def _body(g):
        pltpu.async_copy(table.at[idx_v.at[g]], rows_v, gsem).wait()
        pltpu.sync_copy(rows_v, out.at[pl.ds(base + g * CHUNK, CHUNK)])


def kernel(item_embeddings, batch_data):
    idx = batch_data.astype(jnp.int32).reshape(NUM_WORKERS, NCHUNK, CHUNK)
    mesh = plsc.VectorSubcoreMesh(core_axis_name="c", subcore_axis_name="s")
    flat = pl.kernel(
        _gather_kernel,
        out_type=jax.ShapeDtypeStruct((NUM_IDX, EMBED_DIM), jnp.float32),
        mesh=mesh,
        scratch_types=[
            pltpu.VMEM((NCHUNK, CHUNK), jnp.int32),
            pltpu.VMEM((CHUNK, EMBED_DIM), jnp.float32),
            pltpu.SemaphoreType.DMA,
        ],
        compiler_params=pltpu.CompilerParams(use_tc_tiling_on_sc=False),
    )(item_embeddings, idx)
    return flat.reshape(BATCH, HIST_LEN, EMBED_DIM)


# speedup vs baseline: 1.8766x; 1.1137x over previous
import functools

import jax
import jax.numpy as jnp
from jax import lax
from jax.experimental import pallas as pl
from jax.experimental.pallas import tpu as pltpu
from jax.experimental.pallas import tpu_sc as plsc

VOCAB = 1000000
EMBED_DIM = 64
BATCH = 16384
HIST_LEN = 50

NUM_IDX = BATCH * HIST_LEN
NUM_WORKERS = 32
PER_WORKER = NUM_IDX // NUM_WORKERS
CHUNK = 128
NCHUNK = PER_WORKER // CHUNK


NBUF = 8


def _gather_kernel(table, idx_hbm, out, idx_v, rows_v, gsems, osem):
    wid = lax.axis_index("s") * 2 + lax.axis_index("c")
    base = wid * PER_WORKER
    pltpu.sync_copy(idx_hbm.at[wid], idx_v)

    def start_gather(b, g):
        pltpu.async_copy(table.at[idx_v.at[g]], rows_v.at[b], gsems[b])

    for b in range(NBUF):
        start_gather(b, b)

    @pl.loop(0, NCHUNK, step=NBUF)
    def _body(g0):
        for b in range(NBUF):
            g = g0 + b
            pltpu.make_async_copy(table.at[idx_v.at[g]], rows_v.at[b],
                                  gsems[b]).wait()
            pltpu.async_copy(rows_v.at[b], out.at[pl.ds(base + g * CHUNK, CHUNK)],
                             osem)

            @pl.when(g + NBUF < NCHUNK)
            def _refill():
                pltpu.make_async_copy(
                    rows_v.at[b], out.at[pl.ds(base + g * CHUNK, CHUNK)],
                    osem).wait()
                start_gather(b, g + NBUF)

    for b in range(NBUF):
        g = NCHUNK - NBUF + b
        pltpu.make_async_copy(rows_v.at[b],
                              out.at[pl.ds(base + g * CHUNK, CHUNK)],
                              osem).wait()


def kernel(item_embeddings, batch_data):
    idx = batch_data.astype(jnp.int32).reshape(NUM_WORKERS, NCHUNK, CHUNK)
    mesh = plsc.VectorSubcoreMesh(core_axis_name="c", subcore_axis_name="s")
    flat = pl.kernel(
        _gather_kernel,
        out_type=jax.ShapeDtypeStruct((NUM_IDX, EMBED_DIM), jnp.float32),
        mesh=mesh,
        scratch_types=[
            pltpu.VMEM((NCHUNK, CHUNK), jnp.int32),
            pltpu.VMEM((NBUF, CHUNK, EMBED_DIM), jnp.float32),
            tuple(pltpu.SemaphoreType.DMA for _ in range(NBUF)),
            pltpu.SemaphoreType.DMA,
        ],
        compiler_params=pltpu.CompilerParams(use_tc_tiling_on_sc=False),
    )(item_embeddings, idx)
    return flat.reshape(BATCH, HIST_LEN, EMBED_DIM)
